# baseline (speedup 1.0000x reference)
import functools

import jax
import jax.numpy as jnp
from jax.experimental import pallas as pl
from jax.experimental.pallas import tpu as pltpu

N_BATCH = 16
L = 1024
N_E = 1024
E_DIM = 64
BETA = 0.25
N_ROWS = N_BATCH * L
UNROLL = 2
N_STEPS = N_BATCH // UNROLL


def _vq_kernel(z_ref, mask_ref, emb_ref,
               zq_ref, enc_ref, idx_ref, loss_ref, perp_ref,
               cnt_ref, ssq_ref, emb2_ref, embm2_ref):
    b = pl.program_id(0)

    emb = emb_ref[...]

    @pl.when(b == 0)
    def _init():
        cnt_ref[...] = jnp.zeros_like(cnt_ref)
        ssq_ref[...] = jnp.zeros_like(ssq_ref)
        emb2_ref[...] = jnp.sum(emb * emb, axis=1, keepdims=True).T
        embm2_ref[...] = emb * jnp.float32(-2.0)

    emb2 = emb2_ref[...]
    embm2 = embm2_ref[...]
    ii = jax.lax.broadcasted_iota(jnp.int32, (L, N_E), 1)
    ones_row = jnp.ones((1, L), jnp.float32)

    for k in range(UNROLL):
        z_cl = z_ref[k]
        mask = mask_ref[k]

        zf2 = jnp.sum(z_cl * z_cl, axis=0, keepdims=True)
        zf2_col = zf2.reshape(L, 1)
        mm2 = jax.lax.dot_general(
            z_cl, embm2, (((0,), (1,)), ((), ())),
            preferred_element_type=jnp.float32)
        d = (zf2_col + emb2) + mm2

        dmin = jnp.min(d, axis=1, keepdims=True)
        idx = jnp.min(jnp.where(d == dmin, ii, jnp.int32(N_E)), axis=1,
                      keepdims=True)
        idx_ref[pl.ds(k * L, L), :] = idx

        onehot = (ii == idx).astype(jnp.float32)
        enc_ref[pl.ds(k * L, L), :] = onehot

        zq_cl = jax.lax.dot_general(
            emb, onehot, (((0,), (1,)), ((), ())),
            preferred_element_type=jnp.float32)
        diff = zq_cl - z_cl
        zq_ref[k] = z_cl + diff

        masked = diff * mask
        sq = masked * masked

        cnt_ref[...] += jax.lax.dot_general(
            ones_row, onehot, (((1,), (0,)), ((), ())),
            preferred_element_type=jnp.float32)
        colsq = jax.lax.dot_general(
            jnp.ones((1, E_DIM), jnp.float32), sq, (((1,), (0,)), ((), ())),
            preferred_element_type=jnp.float32)
        ssq_ref[...] += jax.lax.dot_general(
            colsq, jnp.ones((L, 1), jnp.float32), (((1,), (0,)), ((), ())),
            preferred_element_type=jnp.float32)

    @pl.when(b == N_STEPS - 1)
    def _finish():
        c = ssq_ref[...] / jnp.float32(N_ROWS * E_DIM)
        loss_ref[...] = c + jnp.float32(BETA) * c
        e_mean = cnt_ref[...] / jnp.float32(N_ROWS)
        ent = jnp.sum(e_mean * jnp.log(e_mean + 1e-10), axis=(0, 1),
                      keepdims=True)
        perp_ref[...] = jnp.exp(-ent)


@functools.partial(jax.jit, static_argnames=("interpret",))
def kernel(z, mask, emb, interpret=False):
    mask_rows = mask.reshape(N_BATCH, 1, L)

    out_shape = [
        jax.ShapeDtypeStruct((N_BATCH, E_DIM, L), jnp.float32),
        jax.ShapeDtypeStruct((N_ROWS, N_E), jnp.float32),
        jax.ShapeDtypeStruct((N_ROWS, 1), jnp.int32),
        jax.ShapeDtypeStruct((1, 1), jnp.float32),
        jax.ShapeDtypeStruct((1, 1), jnp.float32),
    ]
    z_q_out, enc, idx, loss2, perp2 = pl.pallas_call(
        _vq_kernel,
        grid=(N_STEPS,),
        in_specs=[
            pl.BlockSpec((UNROLL, E_DIM, L), lambda b: (b, 0, 0)),
            pl.BlockSpec((UNROLL, 1, L), lambda b: (b, 0, 0)),
            pl.BlockSpec((N_E, E_DIM), lambda b: (0, 0)),
        ],
        out_specs=[
            pl.BlockSpec((UNROLL, E_DIM, L), lambda b: (b, 0, 0)),
            pl.BlockSpec((UNROLL * L, N_E), lambda b: (b, 0)),
            pl.BlockSpec((UNROLL * L, 1), lambda b: (b, 0)),
            pl.BlockSpec((1, 1), lambda b: (0, 0)),
            pl.BlockSpec((1, 1), lambda b: (0, 0)),
        ],
        out_shape=out_shape,
        scratch_shapes=[
            pltpu.VMEM((1, N_E), jnp.float32),
            pltpu.VMEM((1, 1), jnp.float32),
            pltpu.VMEM((1, N_E), jnp.float32),
            pltpu.VMEM((N_E, E_DIM), jnp.float32),
        ],
        compiler_params=pltpu.CompilerParams(
            dimension_semantics=("arbitrary",)),
        interpret=interpret,
    )(z, mask_rows, emb)

    loss = loss2.reshape(())
    perplexity = perp2.reshape(())
    return (loss, z_q_out, perplexity, enc, idx)


# speedup vs baseline: 1.0070x; 1.0070x over previous
import functools

import jax
import jax.numpy as jnp
from jax.experimental import pallas as pl
from jax.experimental.pallas import tpu as pltpu

N_BATCH = 16
L = 1024
N_E = 1024
E_DIM = 64
BETA = 0.25
N_ROWS = N_BATCH * L


def _vq_kernel(z_ref, mask_ref, emb_ref,
               zq_ref, enc_ref, idx_ref, loss_ref, perp_ref,
               cnt_ref, ssq_ref, emb2_ref, embm2_ref):
    b = pl.program_id(0)

    emb = emb_ref[...]

    @pl.when(b == 0)
    def _init():
        cnt_ref[...] = jnp.zeros_like(cnt_ref)
        ssq_ref[...] = jnp.zeros_like(ssq_ref)
        emb2_ref[...] = jnp.sum(emb * emb, axis=1, keepdims=True).T
        embm2_ref[...] = emb * jnp.float32(-2.0)

    z_cl = z_ref[0]
    mask = mask_ref[0]

    zf2 = jnp.sum(z_cl * z_cl, axis=0, keepdims=True)
    zf2_col = zf2.reshape(L, 1)
    emb2 = emb2_ref[...]
    mm2 = jax.lax.dot_general(z_cl, embm2_ref[...], (((0,), (1,)), ((), ())),
                              preferred_element_type=jnp.float32)
    d = (zf2_col + emb2) + mm2

    dmin = jnp.min(d, axis=1, keepdims=True)
    ii = jax.lax.broadcasted_iota(jnp.int32, (L, N_E), 1)
    idx = jnp.min(jnp.where(d == dmin, ii, jnp.int32(N_E)), axis=1,
                  keepdims=True)
    idx_ref[...] = idx

    onehot = (ii == idx).astype(jnp.float32)
    enc_ref[...] = onehot

    zq_cl = jax.lax.dot_general(emb, onehot, (((0,), (1,)), ((), ())),
                                preferred_element_type=jnp.float32)
    diff = zq_cl - z_cl
    zq_ref[0] = z_cl + diff

    masked = diff * mask
    sq = masked * masked

    ones_row = jnp.ones((1, L), jnp.float32)
    cnt_ref[...] += jax.lax.dot_general(
        ones_row, onehot, (((1,), (0,)), ((), ())),
        preferred_element_type=jnp.float32)
    colsq = jax.lax.dot_general(
        jnp.ones((1, E_DIM), jnp.float32), sq, (((1,), (0,)), ((), ())),
        preferred_element_type=jnp.float32)
    ssq_ref[...] += jax.lax.dot_general(
        colsq, jnp.ones((L, 1), jnp.float32), (((1,), (0,)), ((), ())),
        preferred_element_type=jnp.float32)

    @pl.when(b == N_BATCH - 1)
    def _finish():
        c = ssq_ref[...] / jnp.float32(N_ROWS * E_DIM)
        loss_ref[...] = c + jnp.float32(BETA) * c
        e_mean = cnt_ref[...] / jnp.float32(N_ROWS)
        ent = jnp.sum(e_mean * jnp.log(e_mean + 1e-10), axis=(0, 1),
                      keepdims=True)
        perp_ref[...] = jnp.exp(-ent)


@functools.partial(jax.jit, static_argnames=("interpret",))
def kernel(z, mask, emb, interpret=False):
    mask_rows = mask.reshape(N_BATCH, 1, L)

    out_shape = [
        jax.ShapeDtypeStruct((N_BATCH, E_DIM, L), jnp.float32),
        jax.ShapeDtypeStruct((N_ROWS, N_E), jnp.float32),
        jax.ShapeDtypeStruct((N_ROWS, 1), jnp.int32),
        jax.ShapeDtypeStruct((1, 1), jnp.float32),
        jax.ShapeDtypeStruct((1, 1), jnp.float32),
    ]
    z_q_out, enc, idx, loss2, perp2 = pl.pallas_call(
        _vq_kernel,
        grid=(N_BATCH,),
        in_specs=[
            pl.BlockSpec((1, E_DIM, L), lambda b: (b, 0, 0)),
            pl.BlockSpec((1, 1, L), lambda b: (b, 0, 0)),
            pl.BlockSpec((N_E, E_DIM), lambda b: (0, 0)),
        ],
        out_specs=[
            pl.BlockSpec((1, E_DIM, L), lambda b: (b, 0, 0)),
            pl.BlockSpec((L, N_E), lambda b: (b, 0)),
            pl.BlockSpec((L, 1), lambda b: (b, 0)),
            pl.BlockSpec((1, 1), lambda b: (0, 0)),
            pl.BlockSpec((1, 1), lambda b: (0, 0)),
        ],
        out_shape=out_shape,
        scratch_shapes=[
            pltpu.VMEM((1, N_E), jnp.float32),
            pltpu.VMEM((1, 1), jnp.float32),
            pltpu.VMEM((1, N_E), jnp.float32),
            pltpu.VMEM((N_E, E_DIM), jnp.float32),
        ],
        compiler_params=pltpu.CompilerParams(
            dimension_semantics=("arbitrary",)),
        interpret=interpret,
    )(z, mask_rows, emb)

    loss = loss2.reshape(())
    perplexity = perp2.reshape(())
    return (loss, z_q_out, perplexity, enc, idx)
